# baseline (speedup 1.0000x reference)
import jax
import jax.numpy as jnp
from jax import lax
from jax.experimental import pallas as pl
from jax.experimental.pallas import tpu as pltpu
from jax.experimental.pallas import tpu_sc as plsc

_M, _D, _B = 1000000, 16, 16384
_NT = 16
_SLOTS = _B // _NT
_CHUNK = 128
_NCH = _SLOTS // _CHUNK
_ROWS = (_M // _NT) // 8 * 8
_TAIL = _M - _NT * _ROWS
_DUMMY = _B


def _body(mem, idx, val, zeros, out, s_hbm, accum,
          idx2, w2, weff2, ids2, val_v, cur_v, sums_v, sem, csem):
  t = lax.axis_index("s")
  base = t * _SLOTS

  copy = pltpu.async_copy(mem.at[pl.ds(t * _ROWS, _ROWS)],
                          out.at[pl.ds(t * _ROWS, _ROWS)], csem)

  @pl.when(t == 0)
  def _copy_tail():
    pltpu.async_copy(mem.at[pl.ds(_NT * _ROWS, _TAIL)],
                     out.at[pl.ds(_NT * _ROWS, _TAIL)], csem).wait()

  pltpu.sync_copy(zeros, accum.at[pl.ds(base, _SLOTS)])

  for j in range(_NCH):
    pltpu.sync_copy(idx.at[pl.ds(base + j * _CHUNK, _CHUNK)], idx2.at[j])
  pltpu.sync_copy(val.at[pl.ds(base, _SLOTS)], val_v)
  lanes = lax.iota(jnp.int32, 16)
  for j in range(_NCH):
    for k in range(_CHUNK // 16):
      ids2[j, pl.ds(k * 16, 16)] = lanes + (base + j * _CHUNK + k * 16)

  scat = [pltpu.async_copy(ids2.at[j], s_hbm.at[idx2.at[j]], sem)
          for j in range(_NCH)]
  gcur = [pltpu.async_copy(mem.at[idx2.at[j]],
                           cur_v.at[pl.ds(j * _CHUNK, _CHUNK)], sem)
          for j in range(_NCH)]
  for h in scat + gcur:
    h.wait()
  plsc.subcore_barrier()

  gw = [pltpu.async_copy(s_hbm.at[idx2.at[j]], w2.at[j], sem)
        for j in range(_NCH)]
  for h in gw:
    h.wait()
  for j in range(_NCH):
    for k in range(_CHUNK // 16):
      wv = w2[j, pl.ds(k * 16, 16)]
      iv = ids2[j, pl.ds(k * 16, 16)]
      weff2[j, pl.ds(k * 16, 16)] = jnp.where(wv == iv, wv, _DUMMY)
  addv = [pltpu.async_copy(val_v.at[pl.ds(j * _CHUNK, _CHUNK)],
                           accum.at[w2.at[j]], sem, add=True)
          for j in range(_NCH)]
  addc = [pltpu.async_copy(cur_v.at[pl.ds(j * _CHUNK, _CHUNK)],
                           accum.at[weff2.at[j]], sem, add=True)
          for j in range(_NCH)]
  for h in addv + addc:
    h.wait()
  plsc.subcore_barrier()

  gs = [pltpu.async_copy(accum.at[w2.at[j]],
                         sums_v.at[pl.ds(j * _CHUNK, _CHUNK)], sem)
        for j in range(_NCH)]
  for h in gs:
    h.wait()

  copy.wait()
  plsc.subcore_barrier()

  fin = [pltpu.async_copy(sums_v.at[pl.ds(j * _CHUNK, _CHUNK)],
                          out.at[idx2.at[j]], sem)
         for j in range(_NCH)]
  for h in fin:
    h.wait()


@jax.jit
def _scatter_add(mem, idx, val):
  zeros = jnp.zeros((_SLOTS, _D), jnp.float32)
  mesh = plsc.VectorSubcoreMesh(core_axis_name="c", subcore_axis_name="s",
                                num_cores=1)
  out, _ = pl.kernel(
      _body,
      out_type=(jax.ShapeDtypeStruct((_M, _D), jnp.float32),
                jax.ShapeDtypeStruct((_M,), jnp.int32)),
      mesh=mesh,
      compiler_params=pltpu.CompilerParams(use_tc_tiling_on_sc=False),
      scratch_types=(
          pltpu.VMEM_SHARED((_B + _CHUNK, _D), jnp.float32),
          pltpu.VMEM((_NCH, _CHUNK), jnp.int32),
          pltpu.VMEM((_NCH, _CHUNK), jnp.int32),
          pltpu.VMEM((_NCH, _CHUNK), jnp.int32),
          pltpu.VMEM((_NCH, _CHUNK), jnp.int32),
          pltpu.VMEM((_SLOTS, _D), jnp.float32),
          pltpu.VMEM((_SLOTS, _D), jnp.float32),
          pltpu.VMEM((_SLOTS, _D), jnp.float32),
          pltpu.SemaphoreType.DMA,
          pltpu.SemaphoreType.DMA,
      ),
  )(mem, idx, val, zeros)
  return out


def kernel(mem, idx, val):
  return _scatter_add(mem, idx.astype(jnp.int32), val)


# speedup vs baseline: 1.0021x; 1.0021x over previous
import jax
import jax.numpy as jnp
from jax import lax
from jax.experimental import pallas as pl
from jax.experimental.pallas import tpu as pltpu
from jax.experimental.pallas import tpu_sc as plsc

_M, _D, _B = 1000000, 16, 16384
_NT = 16
_SLOTS = _B // _NT
_CHUNK = 128
_NCH = _SLOTS // _CHUNK
_ROWS = (_M // _NT) // 8 * 8
_TAIL = _M - _NT * _ROWS
_DUMMY = _B


def _body(mem, idx, val, zeros, out, s_hbm, accum,
          idx2, w2, weff2, ids2, val_v, cur_v, sums_v, sem, csem):
  t = lax.axis_index("s")
  base = t * _SLOTS

  _HALF = _ROWS // 2
  copies = [pltpu.async_copy(mem.at[pl.ds(t * _ROWS + c * _HALF, _HALF)],
                             out.at[pl.ds(t * _ROWS + c * _HALF, _HALF)],
                             csem)
            for c in range(2)]

  @pl.when(t == 0)
  def _copy_tail():
    pltpu.async_copy(mem.at[pl.ds(_NT * _ROWS, _TAIL)],
                     out.at[pl.ds(_NT * _ROWS, _TAIL)], csem).wait()

  pltpu.sync_copy(zeros, accum.at[pl.ds(base, _SLOTS)])

  for j in range(_NCH):
    pltpu.sync_copy(idx.at[pl.ds(base + j * _CHUNK, _CHUNK)], idx2.at[j])
  pltpu.sync_copy(val.at[pl.ds(base, _SLOTS)], val_v)
  lanes = lax.iota(jnp.int32, 16)
  for j in range(_NCH):
    for k in range(_CHUNK // 16):
      ids2[j, pl.ds(k * 16, 16)] = lanes + (base + j * _CHUNK + k * 16)

  scat = [pltpu.async_copy(ids2.at[j], s_hbm.at[idx2.at[j]], sem)
          for j in range(_NCH)]
  gcur = [pltpu.async_copy(mem.at[idx2.at[j]],
                           cur_v.at[pl.ds(j * _CHUNK, _CHUNK)], sem)
          for j in range(_NCH)]
  for h in scat + gcur:
    h.wait()
  plsc.subcore_barrier()

  gw = [pltpu.async_copy(s_hbm.at[idx2.at[j]], w2.at[j], sem)
        for j in range(_NCH)]
  for h in gw:
    h.wait()
  for j in range(_NCH):
    for k in range(_CHUNK // 16):
      wv = w2[j, pl.ds(k * 16, 16)]
      iv = ids2[j, pl.ds(k * 16, 16)]
      weff2[j, pl.ds(k * 16, 16)] = jnp.where(wv == iv, wv, _DUMMY)
  addv = [pltpu.async_copy(val_v.at[pl.ds(j * _CHUNK, _CHUNK)],
                           accum.at[w2.at[j]], sem, add=True)
          for j in range(_NCH)]
  addc = [pltpu.async_copy(cur_v.at[pl.ds(j * _CHUNK, _CHUNK)],
                           accum.at[weff2.at[j]], sem, add=True)
          for j in range(_NCH)]
  for h in addv + addc:
    h.wait()
  plsc.subcore_barrier()

  gs = [pltpu.async_copy(accum.at[w2.at[j]],
                         sums_v.at[pl.ds(j * _CHUNK, _CHUNK)], sem)
        for j in range(_NCH)]
  for h in gs:
    h.wait()

  for h in copies:
    h.wait()
  plsc.subcore_barrier()

  fin = [pltpu.async_copy(sums_v.at[pl.ds(j * _CHUNK, _CHUNK)],
                          out.at[idx2.at[j]], sem)
         for j in range(_NCH)]
  for h in fin:
    h.wait()


@jax.jit
def _scatter_add(mem, idx, val):
  zeros = jnp.zeros((_SLOTS, _D), jnp.float32)
  mesh = plsc.VectorSubcoreMesh(core_axis_name="c", subcore_axis_name="s",
                                num_cores=1)
  out, _ = pl.kernel(
      _body,
      out_type=(jax.ShapeDtypeStruct((_M, _D), jnp.float32),
                jax.ShapeDtypeStruct((_M,), jnp.int32)),
      mesh=mesh,
      compiler_params=pltpu.CompilerParams(use_tc_tiling_on_sc=False),
      scratch_types=(
          pltpu.VMEM_SHARED((_B + _CHUNK, _D), jnp.float32),
          pltpu.VMEM((_NCH, _CHUNK), jnp.int32),
          pltpu.VMEM((_NCH, _CHUNK), jnp.int32),
          pltpu.VMEM((_NCH, _CHUNK), jnp.int32),
          pltpu.VMEM((_NCH, _CHUNK), jnp.int32),
          pltpu.VMEM((_SLOTS, _D), jnp.float32),
          pltpu.VMEM((_SLOTS, _D), jnp.float32),
          pltpu.VMEM((_SLOTS, _D), jnp.float32),
          pltpu.SemaphoreType.DMA,
          pltpu.SemaphoreType.DMA,
      ),
  )(mem, idx, val, zeros)
  return out


def kernel(mem, idx, val):
  return _scatter_add(mem, idx.astype(jnp.int32), val)
